# baseline (speedup 1.0000x reference)
import jax
import jax.numpy as jnp
from jax import lax
from jax.experimental import pallas as pl
from jax.experimental.pallas import tpu as pltpu
from jax.experimental.pallas import tpu_sc as plsc

_NC = 2
_NS = 16
_NW = _NC * _NS
_CHUNK = 128
_KC = 5

_D = 16
_NB = 5
_S = 8
_NF = 21
_H = 64
_RPG = 128 // _D


def _gather_body(idx_hbm, locsT_hbm, lscT_hbm, out_l, out_s,
                 gv, outT_l, outT_s, sem):
    wid = lax.axis_index("s") * _NC + lax.axis_index("c")
    pltpu.sync_copy(idx_hbm.at[wid], gv)
    npw = _KC * _CHUNK
    for c in range(_KC):
        cps = []
        for d in range(_D):
            cps.append(pltpu.async_copy(
                locsT_hbm.at[d].at[gv.at[c]],
                outT_l.at[d, pl.ds(c * _CHUNK, _CHUNK)], sem))
            cps.append(pltpu.async_copy(
                lscT_hbm.at[d].at[gv.at[c]],
                outT_s.at[d, pl.ds(c * _CHUNK, _CHUNK)], sem))
        for cp in cps:
            cp.wait()
    pltpu.sync_copy(outT_l, out_l.at[:, pl.ds(wid * npw, npw)])
    pltpu.sync_copy(outT_s, out_s.at[:, pl.ds(wid * npw, npw)])


def _sc_gather(idx3, locsT, lscT, n_rows):
    npw = n_rows // _NW
    f = pl.kernel(
        _gather_body,
        out_type=[
            jax.ShapeDtypeStruct((_D, n_rows), jnp.float32),
            jax.ShapeDtypeStruct((_D, n_rows), jnp.float32),
        ],
        mesh=plsc.VectorSubcoreMesh(core_axis_name="c", subcore_axis_name="s"),
        scratch_types=[
            pltpu.VMEM((_KC, _CHUNK), jnp.int32),
            pltpu.VMEM((_D, npw), jnp.float32),
            pltpu.VMEM((_D, npw), jnp.float32),
            pltpu.SemaphoreType.DMA,
        ],
        compiler_params=pltpu.CompilerParams(use_tc_tiling_on_sc=False),
    )
    return f(idx3, locsT, lscT)


def _dense_body(towers_ref, eps_ref, locs_ref, lsc_ref, w1aT_ref, w1bT_ref,
                b1_ref, w2T_ref, b2_ref, out_ref):
    j = pl.program_id(1)
    t = towers_ref[0]
    lr = locs_ref[:, 0, 0, :]
    sc = jnp.exp(lsc_ref[:, 0, 0, :])
    w1aT = w1aT_ref[...]
    w1bT = w1bT_ref[...]
    b1c = b1_ref[...]
    w2T = w2T_ref[...]
    qx = t[17:18, :]; qy = t[18:19, :]; qz = t[19:20, :]; qw = t[20:21, :]
    s2 = 2.0 / (qx * qx + qy * qy + qz * qz + qw * qw)
    xx = qx * qx * s2; yy = qy * qy * s2; zz = qz * qz * s2
    xy = qx * qy * s2; xz = qx * qz * s2; yz = qy * qz * s2
    xw = qx * qw * s2; yw = qy * qw * s2; zw = qz * qw * s2
    r00 = 1.0 - yy - zz; r01 = xy - zw; r02 = xz + yw
    r10 = xy + zw; r11 = 1.0 - xx - zz; r12 = yz - xw
    r20 = xz - yw; r21 = yz + xw; r22 = 1.0 - xx - yy
    w13 = w1aT[:, 13:14]; w14 = w1aT[:, 14:15]; w15 = w1aT[:, 15:16]
    V0 = w13 * r00 + w14 * r10 + w15 * r20 - w13
    V1 = w13 * r01 + w14 * r11 + w15 * r21 - w14
    V2 = w13 * r02 + w14 * r12 + w15 * r22 - w15
    pre = (jnp.dot(w1bT, t, preferred_element_type=jnp.float32)
           + jnp.dot(w1aT, lr, preferred_element_type=jnp.float32) + b1c
           + lr[13:14, :] * V0 + lr[14:15, :] * V1 + lr[15:16, :] * V2)
    ys = []
    for s in range(_S):
        es = eps_ref[s, 0] * sc
        g = jnp.dot(w1aT, es, preferred_element_type=jnp.float32)
        h = jnp.maximum(
            g + es[13:14, :] * V0 + es[14:15, :] * V1 + es[15:16, :] * V2 + pre,
            0.0)
        ys.append(jnp.dot(w2T, h, preferred_element_type=jnp.float32))
    y = jnp.concatenate(ys, axis=0)

    @pl.when(j == 0)
    def _():
        out_ref[...] = y

    @pl.when(j > 0)
    def _():
        out_ref[...] = out_ref[...] + y

    @pl.when(j == _NB - 1)
    def _():
        out_ref[...] = jax.nn.sigmoid(out_ref[...] * (1.0 / _NB) + b2_ref[...])


def _dense(towers_v, eps_v, locsT, lscT, w1aT, w1bT, b1c, w2T, b2, bb):
    B = towers_v.shape[2]
    return pl.pallas_call(
        _dense_body,
        grid=(B // bb, _NB),
        in_specs=[
            pl.BlockSpec((1, _NF, bb), lambda i, j: (j, 0, i)),
            pl.BlockSpec((_S, 1, _D, bb), lambda i, j: (0, j, 0, i)),
            pl.BlockSpec((_D, 1, 1, bb), lambda i, j: (0, j, 0, i)),
            pl.BlockSpec((_D, 1, 1, bb), lambda i, j: (0, j, 0, i)),
            pl.BlockSpec((_H, _D), lambda i, j: (0, 0)),
            pl.BlockSpec((_H, _NF), lambda i, j: (0, 0)),
            pl.BlockSpec((_H, 1), lambda i, j: (0, 0)),
            pl.BlockSpec((1, _H), lambda i, j: (0, 0)),
            pl.BlockSpec((1, 1), lambda i, j: (0, 0)),
        ],
        out_specs=pl.BlockSpec((_S, bb), lambda i, j: (0, i)),
        out_shape=jax.ShapeDtypeStruct((_S, B), jnp.float32),
    )(towers_v, eps_v, locsT, lscT, w1aT, w1bT, b1c, w2T, b2)


def kernel(towers, block_ids, N_samples, eps, latent_locs, latent_logscales,
           W1, b1, W2, b2):
    B, Nb, Nf = towers.shape
    S = eps.shape[1]
    idx3 = block_ids.T.reshape(_NW, -1, _CHUNK)
    locs_dm, lsc_dm = _sc_gather(idx3, latent_locs.T, latent_logscales.T,
                                 B * Nb)
    locsT = locs_dm.reshape(_D, Nb, 1, B)
    lscT = lsc_dm.reshape(_D, Nb, 1, B)
    y = _dense(towers.transpose(1, 2, 0), eps.transpose(1, 2, 3, 0),
               locsT, lscT, W1[:_D].T, W1[_D:].T,
               b1.reshape(-1, 1), W2.T, b2.reshape(1, 1), bb=1024)
    return y.T[:, None, :]


# speedup vs baseline: 3.0078x; 3.0078x over previous
import jax
import jax.numpy as jnp
from jax import lax
from jax.experimental import pallas as pl
from jax.experimental.pallas import tpu as pltpu
from jax.experimental.pallas import tpu_sc as plsc

_NC = 2
_NS = 16
_NW = _NC * _NS
_CHUNK = 128
_KC = 5

_D = 16
_NB = 5
_S = 8
_NF = 21
_H = 64
_RPG = 128 // _D


def _gather_body(idx_hbm, locs_hbm, lsc_hbm, out_l, out_s,
                 idx_v, gv, blk_l, blk_s, pk_l, pk_s, sem):
    wid = lax.axis_index("s") * _NC + lax.axis_index("c")
    pltpu.sync_copy(idx_hbm.at[wid], idx_v)
    for c in range(_KC):
        for j in range(_CHUNK // 16):
            off = c * _CHUNK + j * 16
            gv[c, pl.ds(j * 16, 16)] = idx_v[pl.ds(off, 16)] >> (_RPG.bit_length() - 1)

    def fire(c, buf):
        cl = pltpu.async_copy(locs_hbm.at[gv.at[c]], blk_l.at[buf], sem)
        cs = pltpu.async_copy(lsc_hbm.at[gv.at[c]], blk_s.at[buf], sem)
        return cl, cs

    lanes = lax.iota(jnp.int32, 16)
    cps = {0: fire(0, 0)}
    for c in range(_KC):
        if c + 1 < _KC:
            cps[c + 1] = fire(c + 1, (c + 1) % 2)
        cps[c][0].wait()
        cps[c][1].wait()
        buf = c % 2

        @pl.loop(0, _CHUNK, unroll=8)
        def select(r):
            g = c * _CHUNK + r
            iv = plsc.load_gather(idx_v, [jnp.full((16,), g, jnp.int32)])
            col = (iv & (_RPG - 1)) * _D + lanes
            row_i = jnp.full((16,), r, jnp.int32)
            o_c = jnp.full((16,), g, jnp.int32)
            vl = plsc.load_gather(blk_l.at[buf], [row_i, col])
            plsc.store_scatter(pk_l, [lanes, o_c], vl)
            vs = plsc.load_gather(blk_s.at[buf], [row_i, col])
            plsc.store_scatter(pk_s, [lanes, o_c], vs)
    npw = _KC * _CHUNK
    pltpu.sync_copy(pk_l, out_l.at[:, pl.ds(wid * npw, npw)])
    pltpu.sync_copy(pk_s, out_s.at[:, pl.ds(wid * npw, npw)])


def _sc_gather(idx2, locs128, lsc128, n_rows):
    npw = n_rows // _NW
    f = pl.kernel(
        _gather_body,
        out_type=[
            jax.ShapeDtypeStruct((_D, n_rows), jnp.float32),
            jax.ShapeDtypeStruct((_D, n_rows), jnp.float32),
        ],
        mesh=plsc.VectorSubcoreMesh(core_axis_name="c", subcore_axis_name="s"),
        scratch_types=[
            pltpu.VMEM((_KC * _CHUNK,), jnp.int32),
            pltpu.VMEM((_KC, _CHUNK), jnp.int32),
            pltpu.VMEM((2, _CHUNK, 128), jnp.float32),
            pltpu.VMEM((2, _CHUNK, 128), jnp.float32),
            pltpu.VMEM((_D, npw), jnp.float32),
            pltpu.VMEM((_D, npw), jnp.float32),
            pltpu.SemaphoreType.DMA,
        ],
        compiler_params=pltpu.CompilerParams(needs_layout_passes=False),
    )
    return f(idx2, locs128, lsc128)


def _dense_body(towers_ref, eps_ref, locs_ref, lsc_ref, w1aT_ref, w1bT_ref,
                b1_ref, w2T_ref, b2_ref, out_ref):
    j = pl.program_id(1)
    t = towers_ref[0]
    lr = locs_ref[:, 0, 0, :]
    sc = jnp.exp(lsc_ref[:, 0, 0, :])
    w1aT = w1aT_ref[...]
    w1bT = w1bT_ref[...]
    b1c = b1_ref[...]
    w2T = w2T_ref[...]
    qx = t[17:18, :]; qy = t[18:19, :]; qz = t[19:20, :]; qw = t[20:21, :]
    s2 = 2.0 / (qx * qx + qy * qy + qz * qz + qw * qw)
    xx = qx * qx * s2; yy = qy * qy * s2; zz = qz * qz * s2
    xy = qx * qy * s2; xz = qx * qz * s2; yz = qy * qz * s2
    xw = qx * qw * s2; yw = qy * qw * s2; zw = qz * qw * s2
    r00 = 1.0 - yy - zz; r01 = xy - zw; r02 = xz + yw
    r10 = xy + zw; r11 = 1.0 - xx - zz; r12 = yz - xw
    r20 = xz - yw; r21 = yz + xw; r22 = 1.0 - xx - yy
    w13 = w1aT[:, 13:14]; w14 = w1aT[:, 14:15]; w15 = w1aT[:, 15:16]
    V0 = w13 * r00 + w14 * r10 + w15 * r20 - w13
    V1 = w13 * r01 + w14 * r11 + w15 * r21 - w14
    V2 = w13 * r02 + w14 * r12 + w15 * r22 - w15
    pre = (jnp.dot(w1bT, t, preferred_element_type=jnp.float32)
           + jnp.dot(w1aT, lr, preferred_element_type=jnp.float32) + b1c
           + lr[13:14, :] * V0 + lr[14:15, :] * V1 + lr[15:16, :] * V2)
    ys = []
    for s in range(_S):
        es = eps_ref[s, 0] * sc
        g = jnp.dot(w1aT, es, preferred_element_type=jnp.float32)
        h = jnp.maximum(
            g + es[13:14, :] * V0 + es[14:15, :] * V1 + es[15:16, :] * V2 + pre,
            0.0)
        ys.append(jnp.dot(w2T, h, preferred_element_type=jnp.float32))
    y = jnp.concatenate(ys, axis=0)

    @pl.when(j == 0)
    def _():
        out_ref[...] = y

    @pl.when(j > 0)
    def _():
        out_ref[...] = out_ref[...] + y

    @pl.when(j == _NB - 1)
    def _():
        out_ref[...] = jax.nn.sigmoid(out_ref[...] * (1.0 / _NB) + b2_ref[...])


def _dense(towers_v, eps_v, locsT, lscT, w1aT, w1bT, b1c, w2T, b2, bb):
    B = towers_v.shape[2]
    return pl.pallas_call(
        _dense_body,
        grid=(B // bb, _NB),
        in_specs=[
            pl.BlockSpec((1, _NF, bb), lambda i, j: (j, 0, i)),
            pl.BlockSpec((_S, 1, _D, bb), lambda i, j: (0, j, 0, i)),
            pl.BlockSpec((_D, 1, 1, bb), lambda i, j: (0, j, 0, i)),
            pl.BlockSpec((_D, 1, 1, bb), lambda i, j: (0, j, 0, i)),
            pl.BlockSpec((_H, _D), lambda i, j: (0, 0)),
            pl.BlockSpec((_H, _NF), lambda i, j: (0, 0)),
            pl.BlockSpec((_H, 1), lambda i, j: (0, 0)),
            pl.BlockSpec((1, _H), lambda i, j: (0, 0)),
            pl.BlockSpec((1, 1), lambda i, j: (0, 0)),
        ],
        out_specs=pl.BlockSpec((_S, bb), lambda i, j: (0, i)),
        out_shape=jax.ShapeDtypeStruct((_S, B), jnp.float32),
    )(towers_v, eps_v, locsT, lscT, w1aT, w1bT, b1c, w2T, b2)


def kernel(towers, block_ids, N_samples, eps, latent_locs, latent_logscales,
           W1, b1, W2, b2):
    B, Nb, Nf = towers.shape
    S = eps.shape[1]
    idx2 = block_ids.T.reshape(_NW, -1)
    locs_rows, lsc_rows = _sc_gather(idx2, latent_locs.reshape(-1, 128),
                                     latent_logscales.reshape(-1, 128), B * Nb)
    locsT = locs_rows.reshape(_D, Nb, 1, B)
    lscT = lsc_rows.reshape(_D, Nb, 1, B)
    y = _dense(towers.transpose(1, 2, 0), eps.transpose(1, 2, 3, 0),
               locsT, lscT, W1[:_D].T, W1[_D:].T,
               b1.reshape(-1, 1), W2.T, b2.reshape(1, 1), bb=1024)
    return y.T[:, None, :]
